# baseline (speedup 1.0000x reference)
import functools

import jax
import jax.numpy as jnp
from jax import lax
from jax.experimental import pallas as pl
from jax.experimental.pallas import tpu as pltpu
from jax.experimental.pallas import tpu_sc as plsc

BATCH = 16384
SEQ_LEN = 50
EMBED_DIM = 64
VOCAB_ROWS = 1000000
TOTAL = BATCH * SEQ_LEN

CHUNK = 128
K = 4
GROUP_ROWS = K * CHUNK
NBUF = 2
NUM_CORES = 2
NUM_SUBCORES = 16
NW = NUM_CORES * NUM_SUBCORES
ROWS_PER_W = TOTAL // NW
CHUNKS_PER_W = ROWS_PER_W // CHUNK
NGROUP = CHUNKS_PER_W // K
NROUND = (NGROUP - NBUF) // NBUF


def _make_kernel():
  mesh = plsc.VectorSubcoreMesh(core_axis_name="c", subcore_axis_name="s")

  @functools.partial(
      pl.kernel,
      mesh=mesh,
      out_type=jax.ShapeDtypeStruct((TOTAL, 2 * EMBED_DIM), jnp.float32),
      compiler_params=pltpu.CompilerParams(use_tc_tiling_on_sc=False),
      scratch_types=[
          pltpu.VMEM((CHUNKS_PER_W, CHUNK), jnp.int32),
          pltpu.VMEM((GROUP_ROWS, EMBED_DIM), jnp.float32),
          pltpu.VMEM((GROUP_ROWS, EMBED_DIM), jnp.float32),
          pltpu.SemaphoreType.DMA,
          pltpu.SemaphoreType.DMA,
          pltpu.SemaphoreType.DMA,
          pltpu.SemaphoreType.DMA,
      ],
  )
  def k(idx_hbm, table_hbm, out_hbm, idx_v, r0, r1, gs0, gs1, os0, os1):
    rows = [r0, r1]
    gsem = [gs0, gs1]
    osem = [os0, os1]
    wid = lax.axis_index("s") * NUM_CORES + lax.axis_index("c")
    chunk0 = wid * CHUNKS_PER_W
    base_row = chunk0 * CHUNK
    pltpu.sync_copy(idx_hbm.at[pl.ds(chunk0, CHUNKS_PER_W)], idx_v)

    def start_group(g, b):
      for kk in range(K):
        pltpu.async_copy(
            table_hbm.at[idx_v.at[g * K + kk]],
            rows[b].at[pl.ds(kk * CHUNK, CHUNK)],
            gsem[b])

    def wait_group(b):
      pltpu.make_async_copy(
          out_hbm.at[pl.ds(0, GROUP_ROWS)], rows[b], gsem[b]).wait()

    def start_out(g, b):
      pltpu.async_copy(
          rows[b],
          out_hbm.at[pl.ds(base_row + g * GROUP_ROWS, GROUP_ROWS),
                     pl.ds(0, EMBED_DIM)],
          osem[b])

    def wait_out(b):
      pltpu.make_async_copy(
          rows[b],
          out_hbm.at[pl.ds(0, GROUP_ROWS), pl.ds(0, EMBED_DIM)],
          osem[b]).wait()

    for b in range(NBUF):
      start_group(b, b)

    def round_body(i, _):
      t = i * NBUF
      for b in range(NBUF):
        wait_group(b)
        start_out(t + b, b)
      for b in range(NBUF):
        wait_out(b)
        start_group(t + NBUF + b, b)
      return 0

    lax.fori_loop(0, NROUND, round_body, 0)

    t = NROUND * NBUF
    for b in range(NBUF):
      wait_group(b)
      start_out(t + b, b)
    for b in range(NBUF):
      wait_out(b)

  return k


_gather = _make_kernel()

TBLK = 512


def _make_table_transpose():
  def body(in_ref, out_ref):
    out_ref[:, 0:EMBED_DIM] = in_ref[...].T

  return pl.pallas_call(
      body,
      grid=(pl.cdiv(VOCAB_ROWS, TBLK),),
      in_specs=[pl.BlockSpec((EMBED_DIM, TBLK), lambda j: (0, j))],
      out_specs=pl.BlockSpec((TBLK, 2 * EMBED_DIM), lambda j: (j, 0)),
      out_shape=jax.ShapeDtypeStruct((VOCAB_ROWS, 2 * EMBED_DIM), jnp.float32),
  )


_table_rowmajor = _make_table_transpose()


def kernel(indices, table):
  idx2d = (indices.T.astype(jnp.int32) * 2).reshape(TOTAL // CHUNK, CHUNK)
  tbl2 = _table_rowmajor(table.T).reshape(2 * VOCAB_ROWS, EMBED_DIM)
  out = _gather(idx2d, tbl2)
  out = out.reshape(SEQ_LEN, BATCH, 2 * EMBED_DIM)[:, :, :EMBED_DIM]
  return out.transpose(1, 0, 2)


# speedup vs baseline: 1.4180x; 1.4180x over previous
import functools

import jax
import jax.numpy as jnp
from jax import lax
from jax.experimental import pallas as pl
from jax.experimental.pallas import tpu as pltpu
from jax.experimental.pallas import tpu_sc as plsc

BATCH = 16384
SEQ_LEN = 50
EMBED_DIM = 64
VOCAB_ROWS = 1000000
TOTAL = BATCH * SEQ_LEN

CHUNK = 128
K = 4
GROUP_ROWS = K * CHUNK
NBUF = 2
NUM_CORES = 2
NUM_SUBCORES = 16
NW = NUM_CORES * NUM_SUBCORES
ROWS_PER_W = TOTAL // NW
CHUNKS_PER_W = ROWS_PER_W // CHUNK
NGROUP = CHUNKS_PER_W // K
NROUND = (NGROUP - NBUF) // NBUF


def _make_kernel():
  mesh = plsc.VectorSubcoreMesh(core_axis_name="c", subcore_axis_name="s")

  @functools.partial(
      pl.kernel,
      mesh=mesh,
      out_type=jax.ShapeDtypeStruct((TOTAL, 2 * EMBED_DIM), jnp.float32),
      compiler_params=pltpu.CompilerParams(use_tc_tiling_on_sc=False),
      scratch_types=[
          pltpu.VMEM((CHUNKS_PER_W, CHUNK), jnp.int32),
          pltpu.VMEM((GROUP_ROWS, EMBED_DIM), jnp.float32),
          pltpu.VMEM((GROUP_ROWS, EMBED_DIM), jnp.float32),
          pltpu.SemaphoreType.DMA,
          pltpu.SemaphoreType.DMA,
          pltpu.SemaphoreType.DMA,
          pltpu.SemaphoreType.DMA,
      ],
  )
  def k(idx_hbm, table_hbm, out_hbm, idx_v, r0, r1, gs0, gs1, os0, os1):
    rows = [r0, r1]
    gsem = [gs0, gs1]
    osem = [os0, os1]
    wid = lax.axis_index("s") * NUM_CORES + lax.axis_index("c")
    chunk0 = wid * CHUNKS_PER_W
    base_row = chunk0 * CHUNK
    pltpu.sync_copy(idx_hbm.at[pl.ds(chunk0, CHUNKS_PER_W)], idx_v)

    def start_group(g, b):
      for kk in range(K):
        pltpu.async_copy(
            table_hbm.at[idx_v.at[g * K + kk]],
            rows[b].at[pl.ds(kk * CHUNK, CHUNK)],
            gsem[b])

    def wait_group(b):
      pltpu.make_async_copy(
          out_hbm.at[pl.ds(0, GROUP_ROWS)], rows[b], gsem[b]).wait()

    def start_out(g, b):
      pltpu.async_copy(
          rows[b],
          out_hbm.at[pl.ds(base_row + g * GROUP_ROWS, GROUP_ROWS),
                     pl.ds(0, EMBED_DIM)],
          osem[b])

    def wait_out(b):
      pltpu.make_async_copy(
          rows[b],
          out_hbm.at[pl.ds(0, GROUP_ROWS), pl.ds(0, EMBED_DIM)],
          osem[b]).wait()

    for b in range(NBUF):
      start_group(b, b)

    def round_body(i, _):
      t = i * NBUF
      for b in range(NBUF):
        wait_group(b)
        start_out(t + b, b)
      for b in range(NBUF):
        wait_out(b)
        start_group(t + NBUF + b, b)
      return 0

    lax.fori_loop(0, NROUND, round_body, 0)

    t = NROUND * NBUF
    for b in range(NBUF):
      wait_group(b)
      start_out(t + b, b)
    for b in range(NBUF):
      wait_out(b)

  return k


_gather = _make_kernel()

TBLK = 1024


def _make_table_transpose():
  def body(in_ref, out_ref):
    eye = jnp.eye(EMBED_DIM, dtype=jnp.float32)
    out_ref[:, 0:EMBED_DIM] = jax.lax.dot_general(
        in_ref[...], eye, (((0,), (0,)), ((), ())),
        preferred_element_type=jnp.float32)

  return pl.pallas_call(
      body,
      grid=(pl.cdiv(VOCAB_ROWS, TBLK),),
      in_specs=[pl.BlockSpec((EMBED_DIM, TBLK), lambda j: (0, j))],
      out_specs=pl.BlockSpec((TBLK, 2 * EMBED_DIM), lambda j: (j, 0)),
      out_shape=jax.ShapeDtypeStruct((VOCAB_ROWS, 2 * EMBED_DIM), jnp.float32),
  )


_table_rowmajor = _make_table_transpose()


def kernel(indices, table):
  idx2d = (indices.T.astype(jnp.int32) * 2).reshape(TOTAL // CHUNK, CHUNK)
  tbl2 = _table_rowmajor(table.T).reshape(2 * VOCAB_ROWS, EMBED_DIM)
  out = _gather(idx2d, tbl2)
  out = out.reshape(SEQ_LEN, BATCH, 2 * EMBED_DIM)[:, :, :EMBED_DIM]
  return out.transpose(1, 0, 2)


# speedup vs baseline: 2.5985x; 1.8325x over previous
import functools

import jax
import jax.numpy as jnp
from jax import lax
from jax.experimental import pallas as pl
from jax.experimental.pallas import tpu as pltpu
from jax.experimental.pallas import tpu_sc as plsc

BATCH = 16384
SEQ_LEN = 50
EMBED_DIM = 64
VOCAB_ROWS = 1000000
TOTAL = BATCH * SEQ_LEN

CHUNK = 128
K = 4
GROUP_ROWS = K * CHUNK
NBUF = 2
NUM_CORES = 2
NUM_SUBCORES = 16
NW = NUM_CORES * NUM_SUBCORES
ROWS_PER_W = TOTAL // NW
CHUNKS_PER_W = ROWS_PER_W // CHUNK
NGROUP = CHUNKS_PER_W // K
NROUND = (NGROUP - NBUF) // NBUF


def _make_kernel():
  mesh = plsc.VectorSubcoreMesh(core_axis_name="c", subcore_axis_name="s")

  @functools.partial(
      pl.kernel,
      mesh=mesh,
      out_type=jax.ShapeDtypeStruct((TOTAL, 2 * EMBED_DIM), jnp.float32),
      compiler_params=pltpu.CompilerParams(use_tc_tiling_on_sc=False),
      scratch_types=[
          pltpu.VMEM((CHUNKS_PER_W, CHUNK), jnp.int32),
          pltpu.VMEM((GROUP_ROWS, EMBED_DIM), jnp.float32),
          pltpu.VMEM((GROUP_ROWS, EMBED_DIM), jnp.float32),
          pltpu.SemaphoreType.DMA,
          pltpu.SemaphoreType.DMA,
          pltpu.SemaphoreType.DMA,
          pltpu.SemaphoreType.DMA,
      ],
  )
  def k(idx_hbm, table_hbm, out_hbm, idx_v, r0, r1, gs0, gs1, os0, os1):
    rows = [r0, r1]
    gsem = [gs0, gs1]
    osem = [os0, os1]
    wid = lax.axis_index("s") * NUM_CORES + lax.axis_index("c")
    chunk0 = wid * CHUNKS_PER_W
    base_row = chunk0 * CHUNK
    pltpu.sync_copy(idx_hbm.at[pl.ds(chunk0, CHUNKS_PER_W)], idx_v)

    def start_group(g, b):
      for kk in range(K):
        pltpu.async_copy(
            table_hbm.at[idx_v.at[g * K + kk]],
            rows[b].at[pl.ds(kk * CHUNK, CHUNK)],
            gsem[b])

    def wait_group(b):
      pltpu.make_async_copy(
          out_hbm.at[pl.ds(0, GROUP_ROWS)], rows[b], gsem[b]).wait()

    def start_out(g, b):
      pltpu.async_copy(
          rows[b],
          out_hbm.at[pl.ds(base_row + g * GROUP_ROWS, GROUP_ROWS),
                     pl.ds(0, EMBED_DIM)],
          osem[b])

    def wait_out(b):
      pltpu.make_async_copy(
          rows[b],
          out_hbm.at[pl.ds(0, GROUP_ROWS), pl.ds(0, EMBED_DIM)],
          osem[b]).wait()

    for b in range(NBUF):
      start_group(b, b)

    def round_body(i, _):
      t = i * NBUF
      for b in range(NBUF):
        wait_group(b)
        start_out(t + b, b)
      for b in range(NBUF):
        wait_out(b)
        start_group(t + NBUF + b, b)
      return 0

    lax.fori_loop(0, NROUND, round_body, 0)

    t = NROUND * NBUF
    for b in range(NBUF):
      wait_group(b)
      start_out(t + b, b)
    for b in range(NBUF):
      wait_out(b)

  return k


_gather = _make_kernel()

TBLK = 8192


def _make_table_transpose():
  def body(in_ref, out_ref):
    eye = jnp.eye(EMBED_DIM, dtype=jnp.float32)
    out_ref[:, 0:EMBED_DIM] = jax.lax.dot_general(
        in_ref[...], eye, (((0,), (0,)), ((), ())),
        preferred_element_type=jnp.float32)

  return pl.pallas_call(
      body,
      grid=(pl.cdiv(VOCAB_ROWS, TBLK),),
      in_specs=[pl.BlockSpec((EMBED_DIM, TBLK), lambda j: (0, j))],
      out_specs=pl.BlockSpec((TBLK, 2 * EMBED_DIM), lambda j: (j, 0)),
      out_shape=jax.ShapeDtypeStruct((VOCAB_ROWS, 2 * EMBED_DIM), jnp.float32),
  )


_table_rowmajor = _make_table_transpose()


def kernel(indices, table):
  idx2d = (indices.T.astype(jnp.int32) * 2).reshape(TOTAL // CHUNK, CHUNK)
  tbl2 = _table_rowmajor(table.T).reshape(2 * VOCAB_ROWS, EMBED_DIM)
  out = _gather(idx2d, tbl2)
  out = out.reshape(SEQ_LEN, BATCH, 2 * EMBED_DIM)[:, :, :EMBED_DIM]
  return out.transpose(1, 0, 2)


# speedup vs baseline: 2.7136x; 1.0443x over previous
import functools

import jax
import jax.numpy as jnp
from jax import lax
from jax.experimental import pallas as pl
from jax.experimental.pallas import tpu as pltpu
from jax.experimental.pallas import tpu_sc as plsc

BATCH = 16384
SEQ_LEN = 50
EMBED_DIM = 64
VOCAB_ROWS = 1000000
TOTAL = BATCH * SEQ_LEN

CHUNK = 128
K = 4
GROUP_ROWS = K * CHUNK
NBUF = 2
NUM_CORES = 2
NUM_SUBCORES = 16
NW = NUM_CORES * NUM_SUBCORES
ROWS_PER_W = TOTAL // NW
CHUNKS_PER_W = ROWS_PER_W // CHUNK
NGROUP = CHUNKS_PER_W // K
NROUND = (NGROUP - NBUF) // NBUF


def _make_kernel():
  mesh = plsc.VectorSubcoreMesh(core_axis_name="c", subcore_axis_name="s")

  @functools.partial(
      pl.kernel,
      mesh=mesh,
      out_type=jax.ShapeDtypeStruct((TOTAL, 2 * EMBED_DIM), jnp.float32),
      compiler_params=pltpu.CompilerParams(use_tc_tiling_on_sc=False),
      scratch_types=[
          pltpu.VMEM((CHUNKS_PER_W, CHUNK), jnp.int32),
          pltpu.VMEM((GROUP_ROWS, EMBED_DIM), jnp.float32),
          pltpu.VMEM((GROUP_ROWS, EMBED_DIM), jnp.float32),
          pltpu.SemaphoreType.DMA,
          pltpu.SemaphoreType.DMA,
          pltpu.SemaphoreType.DMA,
          pltpu.SemaphoreType.DMA,
      ],
  )
  def k(idx_hbm, table_hbm, out_hbm, idx_v, r0, r1, gs0, gs1, os0, os1):
    rows = [r0, r1]
    gsem = [gs0, gs1]
    osem = [os0, os1]
    wid = lax.axis_index("s") * NUM_CORES + lax.axis_index("c")
    chunk0 = wid * CHUNKS_PER_W
    base_row = chunk0 * CHUNK
    pltpu.sync_copy(idx_hbm.at[pl.ds(chunk0, CHUNKS_PER_W)], idx_v)

    def start_group(g, b):
      for kk in range(K):
        pltpu.async_copy(
            table_hbm.at[idx_v.at[g * K + kk]],
            rows[b].at[pl.ds(kk * CHUNK, CHUNK)],
            gsem[b])

    def wait_group(b):
      pltpu.make_async_copy(
          out_hbm.at[pl.ds(0, GROUP_ROWS)], rows[b], gsem[b]).wait()

    def start_out(g, b):
      pltpu.async_copy(
          rows[b],
          out_hbm.at[pl.ds(base_row + g * GROUP_ROWS, GROUP_ROWS),
                     pl.ds(0, EMBED_DIM)],
          osem[b])

    def wait_out(b):
      pltpu.make_async_copy(
          rows[b],
          out_hbm.at[pl.ds(0, GROUP_ROWS), pl.ds(0, EMBED_DIM)],
          osem[b]).wait()

    for b in range(NBUF):
      start_group(b, b)

    def round_body(i, _):
      t = i * NBUF
      for b in range(NBUF):
        wait_group(b)
        start_out(t + b, b)
      for b in range(NBUF):
        wait_out(b)
        start_group(t + NBUF + b, b)
      return 0

    lax.fori_loop(0, NROUND, round_body, 0)

    t = NROUND * NBUF
    for b in range(NBUF):
      wait_group(b)
      start_out(t + b, b)
    for b in range(NBUF):
      wait_out(b)

  return k


_gather = _make_kernel()

TBLK = 16384


def _make_table_transpose():
  def body(in_ref, out_ref):
    eye = jnp.eye(EMBED_DIM, dtype=jnp.float32)
    out_ref[:, 0:EMBED_DIM] = jax.lax.dot_general(
        in_ref[...], eye, (((0,), (0,)), ((), ())),
        preferred_element_type=jnp.float32)

  return pl.pallas_call(
      body,
      grid=(pl.cdiv(VOCAB_ROWS, TBLK),),
      in_specs=[pl.BlockSpec((EMBED_DIM, TBLK), lambda j: (0, j))],
      out_specs=pl.BlockSpec((TBLK, 2 * EMBED_DIM), lambda j: (j, 0)),
      out_shape=jax.ShapeDtypeStruct((VOCAB_ROWS, 2 * EMBED_DIM), jnp.float32),
  )


_table_rowmajor = _make_table_transpose()


def kernel(indices, table):
  idx2d = (indices.T.astype(jnp.int32) * 2).reshape(TOTAL // CHUNK, CHUNK)
  tbl2 = _table_rowmajor(table.T).reshape(2 * VOCAB_ROWS, EMBED_DIM)
  out = _gather(idx2d, tbl2)
  out = out.reshape(SEQ_LEN, BATCH, 2 * EMBED_DIM)[:, :, :EMBED_DIM]
  return out.transpose(1, 0, 2)


# speedup vs baseline: 2.7275x; 1.0051x over previous
import functools

import jax
import jax.numpy as jnp
from jax import lax
from jax.experimental import pallas as pl
from jax.experimental.pallas import tpu as pltpu
from jax.experimental.pallas import tpu_sc as plsc

BATCH = 16384
SEQ_LEN = 50
EMBED_DIM = 64
VOCAB_ROWS = 1000000
TOTAL = BATCH * SEQ_LEN

CHUNK = 128
K = 2
GROUP_ROWS = K * CHUNK
NBUF = 4
NUM_CORES = 2
NUM_SUBCORES = 16
NW = NUM_CORES * NUM_SUBCORES
ROWS_PER_W = TOTAL // NW
CHUNKS_PER_W = ROWS_PER_W // CHUNK
NGROUP = CHUNKS_PER_W // K
NROUND = (NGROUP - NBUF) // NBUF


def _make_kernel():
  mesh = plsc.VectorSubcoreMesh(core_axis_name="c", subcore_axis_name="s")

  @functools.partial(
      pl.kernel,
      mesh=mesh,
      out_type=jax.ShapeDtypeStruct((TOTAL, 2 * EMBED_DIM), jnp.float32),
      compiler_params=pltpu.CompilerParams(use_tc_tiling_on_sc=False),
      scratch_types=[
          pltpu.VMEM((CHUNKS_PER_W, CHUNK), jnp.int32),
      ] + [pltpu.VMEM((GROUP_ROWS, EMBED_DIM), jnp.float32)] * NBUF
        + [pltpu.SemaphoreType.DMA] * (2 * NBUF),
  )
  def k(idx_hbm, table_hbm, out_hbm, idx_v, *bufs):
    rows = list(bufs[:NBUF])
    gsem = list(bufs[NBUF:2 * NBUF])
    osem = list(bufs[2 * NBUF:])
    wid = lax.axis_index("s") * NUM_CORES + lax.axis_index("c")
    chunk0 = wid * CHUNKS_PER_W
    base_row = chunk0 * CHUNK
    pltpu.sync_copy(idx_hbm.at[pl.ds(chunk0, CHUNKS_PER_W)], idx_v)

    def start_group(g, b):
      for kk in range(K):
        pltpu.async_copy(
            table_hbm.at[idx_v.at[g * K + kk]],
            rows[b].at[pl.ds(kk * CHUNK, CHUNK)],
            gsem[b])

    def wait_group(b):
      pltpu.make_async_copy(
          out_hbm.at[pl.ds(0, GROUP_ROWS)], rows[b], gsem[b]).wait()

    def start_out(g, b):
      pltpu.async_copy(
          rows[b],
          out_hbm.at[pl.ds(base_row + g * GROUP_ROWS, GROUP_ROWS),
                     pl.ds(0, EMBED_DIM)],
          osem[b])

    def wait_out(b):
      pltpu.make_async_copy(
          rows[b],
          out_hbm.at[pl.ds(0, GROUP_ROWS), pl.ds(0, EMBED_DIM)],
          osem[b]).wait()

    for b in range(NBUF):
      start_group(b, b)

    def round_body(i, _):
      t = i * NBUF
      for b in range(NBUF):
        wait_group(b)
        start_out(t + b, b)
      for b in range(NBUF):
        wait_out(b)
        start_group(t + NBUF + b, b)
      return 0

    lax.fori_loop(0, NROUND, round_body, 0)

    t = NROUND * NBUF
    for b in range(NBUF):
      wait_group(b)
      start_out(t + b, b)
    for b in range(NBUF):
      wait_out(b)

  return k


_gather = _make_kernel()

TBLK = 16384


def _make_table_transpose():
  def body(in_ref, out_ref):
    eye = jnp.eye(EMBED_DIM, dtype=jnp.float32)
    out_ref[:, 0:EMBED_DIM] = jax.lax.dot_general(
        in_ref[...], eye, (((0,), (0,)), ((), ())),
        preferred_element_type=jnp.float32)

  return pl.pallas_call(
      body,
      grid=(pl.cdiv(VOCAB_ROWS, TBLK),),
      in_specs=[pl.BlockSpec((EMBED_DIM, TBLK), lambda j: (0, j))],
      out_specs=pl.BlockSpec((TBLK, 2 * EMBED_DIM), lambda j: (j, 0)),
      out_shape=jax.ShapeDtypeStruct((VOCAB_ROWS, 2 * EMBED_DIM), jnp.float32),
  )


_table_rowmajor = _make_table_transpose()


def kernel(indices, table):
  idx2d = (indices.T.astype(jnp.int32) * 2).reshape(TOTAL // CHUNK, CHUNK)
  tbl2 = _table_rowmajor(table.T).reshape(2 * VOCAB_ROWS, EMBED_DIM)
  out = _gather(idx2d, tbl2)
  out = out.reshape(SEQ_LEN, BATCH, 2 * EMBED_DIM)[:, :, :EMBED_DIM]
  return out.transpose(1, 0, 2)


# speedup vs baseline: 2.7580x; 1.0112x over previous
import functools

import jax
import jax.numpy as jnp
from jax import lax
from jax.experimental import pallas as pl
from jax.experimental.pallas import tpu as pltpu
from jax.experimental.pallas import tpu_sc as plsc

BATCH = 16384
SEQ_LEN = 50
EMBED_DIM = 64
VOCAB_ROWS = 1000000
TOTAL = BATCH * SEQ_LEN

CHUNK = 128
K = 2
GROUP_ROWS = K * CHUNK
NBUF = 4
NUM_CORES = 2
NUM_SUBCORES = 16
NW = NUM_CORES * NUM_SUBCORES
ROWS_PER_W = TOTAL // NW
CHUNKS_PER_W = ROWS_PER_W // CHUNK
NGROUP = CHUNKS_PER_W // K
NROUND = (NGROUP - NBUF) // NBUF


def _make_kernel():
  mesh = plsc.VectorSubcoreMesh(core_axis_name="c", subcore_axis_name="s")

  @functools.partial(
      pl.kernel,
      mesh=mesh,
      out_type=jax.ShapeDtypeStruct((TOTAL, 2 * EMBED_DIM), jnp.float32),
      compiler_params=pltpu.CompilerParams(use_tc_tiling_on_sc=False),
      scratch_types=[
          pltpu.VMEM((CHUNKS_PER_W, CHUNK), jnp.int32),
      ] + [pltpu.VMEM((GROUP_ROWS, EMBED_DIM), jnp.float32)] * NBUF
        + [pltpu.SemaphoreType.DMA] * (2 * NBUF),
  )
  def k(idx_hbm, table_hbm, out_hbm, idx_v, *bufs):
    rows = list(bufs[:NBUF])
    gsem = list(bufs[NBUF:2 * NBUF])
    osem = list(bufs[2 * NBUF:])
    wid = lax.axis_index("s") * NUM_CORES + lax.axis_index("c")
    chunk0 = wid * CHUNKS_PER_W
    base_row = chunk0 * CHUNK
    pltpu.sync_copy(idx_hbm.at[pl.ds(chunk0, CHUNKS_PER_W)], idx_v)

    def start_group(g, b):
      for kk in range(K):
        pltpu.async_copy(
            table_hbm.at[idx_v.at[g * K + kk]],
            rows[b].at[pl.ds(kk * CHUNK, CHUNK)],
            gsem[b])

    def wait_group(b):
      pltpu.make_async_copy(
          out_hbm.at[pl.ds(0, GROUP_ROWS)], rows[b], gsem[b]).wait()

    def start_out(g, b):
      pltpu.async_copy(
          rows[b],
          out_hbm.at[pl.ds(base_row + g * GROUP_ROWS, GROUP_ROWS),
                     pl.ds(0, EMBED_DIM)],
          osem[b])

    def wait_out(b):
      pltpu.make_async_copy(
          rows[b],
          out_hbm.at[pl.ds(0, GROUP_ROWS), pl.ds(0, EMBED_DIM)],
          osem[b]).wait()

    for b in range(NBUF):
      start_group(b, b)

    def round_body(i, _):
      t = i * NBUF
      for b in range(NBUF):
        wait_group(b)
        start_out(t + b, b)
      for b in range(NBUF):
        wait_out(b)
        start_group(t + NBUF + b, b)
      return 0

    lax.fori_loop(0, NROUND, round_body, 0)

    t = NROUND * NBUF
    for b in range(NBUF):
      wait_group(b)
      start_out(t + b, b)
    for b in range(NBUF):
      wait_out(b)

  return k


_gather = _make_kernel()

TBLK = 32768


def _make_table_transpose():
  def body(in_ref, out_ref):
    eye = jnp.eye(EMBED_DIM, dtype=jnp.float32)
    out_ref[:, 0:EMBED_DIM] = jax.lax.dot_general(
        in_ref[...], eye, (((0,), (0,)), ((), ())),
        preferred_element_type=jnp.float32)

  return pl.pallas_call(
      body,
      grid=(pl.cdiv(VOCAB_ROWS, TBLK),),
      in_specs=[pl.BlockSpec((EMBED_DIM, TBLK), lambda j: (0, j))],
      out_specs=pl.BlockSpec((TBLK, 2 * EMBED_DIM), lambda j: (j, 0)),
      out_shape=jax.ShapeDtypeStruct((VOCAB_ROWS, 2 * EMBED_DIM), jnp.float32),
  )


_table_rowmajor = _make_table_transpose()


def kernel(indices, table):
  idx2d = (indices.T.astype(jnp.int32) * 2).reshape(TOTAL // CHUNK, CHUNK)
  tbl2 = _table_rowmajor(table.T).reshape(2 * VOCAB_ROWS, EMBED_DIM)
  out = _gather(idx2d, tbl2)
  out = out.reshape(SEQ_LEN, BATCH, 2 * EMBED_DIM)[:, :, :EMBED_DIM]
  return out.transpose(1, 0, 2)
